# Initial kernel scaffold; baseline (speedup 1.0000x reference)
import jax
import jax.numpy as jnp
from jax.experimental import pallas as pl


def kernel(rules, weight, bias):
    raise NotImplementedError("write your pallas kernel here")


# speedup vs baseline: 246.6344x; 246.6344x over previous
import functools

import jax
import jax.numpy as jnp
from jax import lax
from jax.experimental import pallas as pl
from jax.experimental.pallas import tpu as pltpu
from jax.experimental.pallas import tpu_sc as plsc

BATCH = 16384
HIST = 200
TABLE_PAD = 1024

NC = 2
NS = 16
LANES = 16
NW = NC * NS

ROWS_PER_W = BATCH // NW
CHUNK_ROWS = 64
CHUNK_ELEMS = CHUNK_ROWS * HIST
NCHUNKS = ROWS_PER_W // CHUNK_ROWS
GROUPS_PER_CHUNK = CHUNK_ROWS // LANES


def _sc_kernel(rules_hbm, table_hbm, bias_hbm, out_hbm,
               table_v, bias_v, buf0, buf1, out_v, sem0, sem1):
    wid = lax.axis_index("s") * NC + lax.axis_index("c")
    row_base = wid * ROWS_PER_W
    elem_base = row_base * HIST

    pltpu.sync_copy(table_hbm, table_v)
    pltpu.sync_copy(bias_hbm, bias_v)

    bufs = (buf0, buf1)
    sems = (sem0, sem1)

    def start(c):
        pltpu.async_copy(
            rules_hbm.at[pl.ds(elem_base + c * CHUNK_ELEMS, CHUNK_ELEMS)],
            bufs[c % 2], sems[c % 2])

    start(0)

    lane_offs = lax.iota(jnp.int32, LANES) * HIST

    for c in range(NCHUNKS):
        buf = bufs[c % 2]
        pltpu.make_async_copy(
            rules_hbm.at[pl.ds(elem_base + c * CHUNK_ELEMS, CHUNK_ELEMS)],
            buf, sems[c % 2]).wait()
        if c + 1 < NCHUNKS:
            start(c + 1)

        for g in range(GROUPS_PER_CHUNK):
            base_offs = lane_offs + g * (LANES * HIST)

            def body(l, acc):
                idx = plsc.load_gather(buf, [base_offs + l])
                vals = plsc.load_gather(table_v, [idx])
                return acc + vals

            acc = lax.fori_loop(0, HIST, body, bias_v[...])
            out_v[pl.ds((c * GROUPS_PER_CHUNK + g) * LANES, LANES)] = acc

    pltpu.sync_copy(out_v, out_hbm.at[pl.ds(row_base, ROWS_PER_W)])


@jax.jit
def _run(rules_flat, table_pad, bias16):
    mesh = plsc.VectorSubcoreMesh(
        core_axis_name="c", subcore_axis_name="s",
        num_cores=NC, num_subcores=NS)
    f = pl.kernel(
        _sc_kernel,
        out_type=jax.ShapeDtypeStruct((BATCH,), jnp.float32),
        mesh=mesh,
        scratch_types=[
            pltpu.VMEM((TABLE_PAD,), jnp.float32),
            pltpu.VMEM((LANES,), jnp.float32),
            pltpu.VMEM((CHUNK_ELEMS,), jnp.int32),
            pltpu.VMEM((CHUNK_ELEMS,), jnp.int32),
            pltpu.VMEM((ROWS_PER_W,), jnp.float32),
            pltpu.SemaphoreType.DMA,
            pltpu.SemaphoreType.DMA,
        ],
        compiler_params=pltpu.CompilerParams(needs_layout_passes=False),
    )
    return f(rules_flat, table_pad, bias16)


def kernel(rules, weight, bias):
    rules_flat = rules.reshape(-1).astype(jnp.int32)
    table_pad = jnp.pad(weight.reshape(-1), (0, TABLE_PAD - weight.shape[0]))
    bias16 = jnp.broadcast_to(bias.reshape(1), (LANES,))
    out = _run(rules_flat, table_pad, bias16)
    return out.reshape(BATCH, 1)


# speedup vs baseline: 360.2982x; 1.4609x over previous
import functools

import jax
import jax.numpy as jnp
from jax import lax
from jax.experimental import pallas as pl
from jax.experimental.pallas import tpu as pltpu
from jax.experimental.pallas import tpu_sc as plsc

BATCH = 16384
HIST = 200
TABLE_PAD = 1024

NC = 2
NS = 16
LANES = 16
NW = NC * NS

ROWS_PER_W = BATCH // NW
CHUNK_ROWS = 64
CHUNK_ELEMS = CHUNK_ROWS * HIST
NCHUNKS = ROWS_PER_W // CHUNK_ROWS
GROUPS_PER_CHUNK = CHUNK_ROWS // LANES


def _sc_kernel(rules_hbm, table_hbm, bias_hbm, out_hbm,
               table_v, bias_v, buf0, buf1, out_v, sem0, sem1):
    wid = lax.axis_index("s") * NC + lax.axis_index("c")
    row_base = wid * ROWS_PER_W
    elem_base = row_base * HIST

    pltpu.sync_copy(table_hbm, table_v)
    pltpu.sync_copy(bias_hbm, bias_v)

    bufs = (buf0, buf1)
    sems = (sem0, sem1)

    def start(c):
        pltpu.async_copy(
            rules_hbm.at[pl.ds(elem_base + c * CHUNK_ELEMS, CHUNK_ELEMS)],
            bufs[c % 2], sems[c % 2])

    start(0)

    lane_offs = lax.iota(jnp.int32, LANES) * HIST

    for c in range(NCHUNKS):
        buf = bufs[c % 2]
        pltpu.make_async_copy(
            rules_hbm.at[pl.ds(elem_base + c * CHUNK_ELEMS, CHUNK_ELEMS)],
            buf, sems[c % 2]).wait()
        if c + 1 < NCHUNKS:
            start(c + 1)

        def group_body(g, _):
            base_offs = lane_offs + g * (LANES * HIST)
            zero = jnp.zeros((LANES,), jnp.float32)

            @plsc.parallel_loop(0, HIST // 4, 1, unroll=4,
                                carry=(zero, zero, zero, zero))
            def accs(l, carry):
                a0, a1, a2, a3 = carry
                q = HIST // 4
                i0 = plsc.load_gather(buf, [base_offs + l])
                i1 = plsc.load_gather(buf, [base_offs + (l + q)])
                i2 = plsc.load_gather(buf, [base_offs + (l + 2 * q)])
                i3 = plsc.load_gather(buf, [base_offs + (l + 3 * q)])
                a0 = a0 + plsc.load_gather(table_v, [i0])
                a1 = a1 + plsc.load_gather(table_v, [i1])
                a2 = a2 + plsc.load_gather(table_v, [i2])
                a3 = a3 + plsc.load_gather(table_v, [i3])
                return a0, a1, a2, a3

            acc = (accs[0] + accs[1]) + (accs[2] + accs[3]) + bias_v[...]
            out_v[pl.ds((c * GROUPS_PER_CHUNK + g) * LANES, LANES)] = acc
            return 0

        lax.fori_loop(0, GROUPS_PER_CHUNK, group_body, 0)

    pltpu.sync_copy(out_v, out_hbm.at[pl.ds(row_base, ROWS_PER_W)])


@jax.jit
def _run(rules_flat, table_pad, bias16):
    mesh = plsc.VectorSubcoreMesh(
        core_axis_name="c", subcore_axis_name="s",
        num_cores=NC, num_subcores=NS)
    f = pl.kernel(
        _sc_kernel,
        out_type=jax.ShapeDtypeStruct((BATCH,), jnp.float32),
        mesh=mesh,
        scratch_types=[
            pltpu.VMEM((TABLE_PAD,), jnp.float32),
            pltpu.VMEM((LANES,), jnp.float32),
            pltpu.VMEM((CHUNK_ELEMS,), jnp.int32),
            pltpu.VMEM((CHUNK_ELEMS,), jnp.int32),
            pltpu.VMEM((ROWS_PER_W,), jnp.float32),
            pltpu.SemaphoreType.DMA,
            pltpu.SemaphoreType.DMA,
        ],
        compiler_params=pltpu.CompilerParams(needs_layout_passes=False),
    )
    return f(rules_flat, table_pad, bias16)


def kernel(rules, weight, bias):
    rules_flat = rules.reshape(-1).astype(jnp.int32)
    table_pad = jnp.pad(weight.reshape(-1), (0, TABLE_PAD - weight.shape[0]))
    bias16 = jnp.broadcast_to(bias.reshape(1), (LANES,))
    out = _run(rules_flat, table_pad, bias16)
    return out.reshape(BATCH, 1)
